# Initial kernel scaffold; baseline (speedup 1.0000x reference)
import jax
import jax.numpy as jnp
from jax.experimental import pallas as pl


def kernel(x, adj, x_g_b, W_fc, bias_gc, prelu_a, W_bil, b_bil):
    raise NotImplementedError("write your pallas kernel here")


# speedup vs baseline: 1.9554x; 1.9554x over previous
import jax
import jax.numpy as jnp
from functools import partial
from jax.experimental import pallas as pl
from jax.experimental.pallas import tpu as pltpu


def _fused_kernel(x_ref, adj_ref, xgb_ref, wfcT_ref, wbil_ref, bias_ref,
                  a_ref, b_ref, out_ref, cv_scr, *, BS, S, F, H, B, NB):
    i = pl.program_id(0)

    x2 = x_ref[...]
    wfcT = wfcT_ref[...]
    adj2 = adj_ref[...]
    bias = bias_ref[...]
    a = a_ref[...]

    seq = [jnp.dot(x2[:, j * F:(j + 1) * F], wfcT,
                   preferred_element_type=jnp.float32) for j in range(S)]

    hs = []
    for ii in range(S):
        acc = seq[0] * adj2[:, ii * S:ii * S + 1]
        for j in range(1, S):
            acc = acc + seq[j] * adj2[:, ii * S + j:ii * S + j + 1]
        o = acc + bias
        t = jnp.where(o >= 0, o, a * o)
        hs.append(jnp.maximum(t, 0.0))

    c = hs[0]
    for j in range(1, S - 1):
        c = c + hs[j]
    c = c * (1.0 / (S - 1))
    hmv = 0.5 * hs[S - 1] + xgb_ref[...]
    v = jnp.dot(hmv, wbil_ref[...], preferred_element_type=jnp.float32)

    cv_scr[pl.ds(i * BS, BS), :] = jnp.concatenate([c, v], axis=1)

    @pl.when(i == NB - 1)
    def _():
        CV = cv_scr[...]
        C = CV[:, :H]
        V = CV[:, H:]
        s1 = jnp.sum(V * C, axis=1, keepdims=True)
        Cr = jnp.concatenate([CV[B - 1:, :H], CV[:B - 1, :H]], axis=0)
        s2 = jnp.sum(V * Cr, axis=1, keepdims=True)
        out_ref[...] = jnp.concatenate([s1, s2], axis=1) + b_ref[...]


@jax.jit
def kernel(x, adj, x_g_b, W_fc, bias_gc, prelu_a, W_bil, b_bil):
    B, S, F = x.shape
    H = W_fc.shape[0]
    BS = 2048
    NB = B // BS

    x2d = x.reshape(B, S * F)
    adj2d = adj.reshape(B, S * S)
    wfcT = W_fc.T
    wbil = W_bil.reshape(H, H)
    bias2 = bias_gc.reshape(1, H)
    a2 = jnp.reshape(prelu_a, (1, 1)).astype(jnp.float32)
    b2 = jnp.reshape(b_bil, (1, 1)).astype(jnp.float32)

    body = partial(_fused_kernel, BS=BS, S=S, F=F, H=H, B=B, NB=NB)
    out = pl.pallas_call(
        body,
        grid=(NB,),
        in_specs=[
            pl.BlockSpec((BS, S * F), lambda i: (i, 0)),
            pl.BlockSpec((BS, S * S), lambda i: (i, 0)),
            pl.BlockSpec((BS, H), lambda i: (i, 0)),
            pl.BlockSpec((F, H), lambda i: (0, 0)),
            pl.BlockSpec((H, H), lambda i: (0, 0)),
            pl.BlockSpec((1, H), lambda i: (0, 0)),
            pl.BlockSpec((1, 1), lambda i: (0, 0)),
            pl.BlockSpec((1, 1), lambda i: (0, 0)),
        ],
        out_specs=pl.BlockSpec((B, 2), lambda i: (0, 0)),
        out_shape=jax.ShapeDtypeStruct((B, 2), jnp.float32),
        scratch_shapes=[pltpu.VMEM((B, 2 * H), jnp.float32)],
    )(x2d, adj2d, x_g_b, wfcT, wbil, bias2, a2, b2)

    return out.T.reshape(2 * B, 1)
